# baseline (speedup 1.0000x reference)
import jax
import jax.numpy as jnp
from jax.experimental import pallas as pl
from jax.experimental.pallas import tpu as pltpu

E = 64
K = 8
R = 16
TEMP = 0.5
EPS = 1e-6
TILE = 512


def _phase1_kernel(x_ref, ws_ref, h_ref, pmax_ref):
    h = jax.lax.dot_general(
        x_ref[:], ws_ref[:],
        dimension_numbers=(((1,), (1,)), ((), ())),
        preferred_element_type=jnp.float32,
    )
    h_ref[:] = h
    pmax_ref[:] = jnp.max(jnp.abs(h)).reshape(1, 1, 1)


def _phase2_kernel(x_ref, h_ref, pmax_ref, w_ref, b_ref, a_ref, bm_ref,
                   limes_ref, out_ref):
    x = x_ref[:].astype(jnp.bfloat16)
    base = jax.lax.dot_general(
        x, w_ref[:],
        dimension_numbers=(((1,), (1,)), ((), ())),
        preferred_element_type=jnp.float32,
    ) + b_ref[:]

    scale = jnp.maximum(jnp.max(pmax_ref[:]), EPS)
    inv = 1.0 / (scale * TEMP)
    logits = h_ref[:] * inv
    m = jnp.max(logits, axis=-1, keepdims=True)
    e = jnp.exp(logits - m)
    ii = jax.lax.broadcasted_iota(jnp.int32, e.shape, 1)

    masked = e
    wmat = jnp.zeros_like(e)
    ssum = jnp.zeros((e.shape[0], 1), jnp.float32)
    for _ in range(K):
        cur = jnp.max(masked, axis=-1, keepdims=True)
        ismax = masked == cur
        midx = jnp.where(ismax, ii, E)
        first = midx == jnp.min(midx, axis=-1, keepdims=True)
        wmat = wmat + jnp.where(first, masked, 0.0)
        ssum = ssum + cur
        masked = jnp.where(first, -1.0, masked)

    w = wmat / ssum
    p_mix = jnp.dot(w, limes_ref[:], preferred_element_type=jnp.float32)

    u = jnp.dot(x, a_ref[:], preferred_element_type=jnp.float32)
    mod = (u * p_mix).astype(jnp.bfloat16)
    delta = jnp.dot(mod, bm_ref[:], preferred_element_type=jnp.float32)
    out_ref[:] = base + delta


def kernel(x, W, b, A, Bm, LiMEs):
    B, T, d_in = x.shape
    d_out = W.shape[0]
    n_tok = B * T
    nt = n_tok // TILE
    x2 = x.reshape(n_tok, d_in)
    W_bf = W.astype(jnp.bfloat16)
    A_bf = A.astype(jnp.bfloat16)
    Bm_bf = Bm.astype(jnp.bfloat16)

    h, pmax = pl.pallas_call(
        _phase1_kernel,
        grid=(nt,),
        in_specs=[
            pl.BlockSpec((TILE, d_in), lambda i: (i, 0)),
            pl.BlockSpec((E, d_in), lambda i: (0, 0)),
        ],
        out_specs=[
            pl.BlockSpec((TILE, E), lambda i: (i, 0)),
            pl.BlockSpec((1, 1, 1), lambda i: (i, 0, 0)),
        ],
        out_shape=[
            jax.ShapeDtypeStruct((n_tok, E), jnp.float32),
            jax.ShapeDtypeStruct((nt, 1, 1), jnp.float32),
        ],
        compiler_params=pltpu.CompilerParams(
            dimension_semantics=("parallel",)),
    )(x2, W)

    out = pl.pallas_call(
        _phase2_kernel,
        grid=(nt,),
        in_specs=[
            pl.BlockSpec((TILE, d_in), lambda i: (i, 0)),
            pl.BlockSpec((TILE, E), lambda i: (i, 0)),
            pl.BlockSpec((nt, 1, 1), lambda i: (0, 0, 0)),
            pl.BlockSpec((d_out, d_in), lambda i: (0, 0)),
            pl.BlockSpec((1, d_out), lambda i: (0, 0)),
            pl.BlockSpec((d_in, R), lambda i: (0, 0)),
            pl.BlockSpec((R, d_out), lambda i: (0, 0)),
            pl.BlockSpec((E, R), lambda i: (0, 0)),
        ],
        out_specs=pl.BlockSpec((TILE, d_out), lambda i: (i, 0)),
        out_shape=jax.ShapeDtypeStruct((n_tok, d_out), jnp.float32),
        compiler_params=pltpu.CompilerParams(
            dimension_semantics=("parallel",)),
    )(x2, h, pmax, W_bf, b.reshape(1, d_out), A_bf, Bm_bf, LiMEs)

    return out.reshape(B, T, d_out)


# speedup vs baseline: 1.1560x; 1.1560x over previous
import jax
import jax.numpy as jnp
from jax.experimental import pallas as pl
from jax.experimental.pallas import tpu as pltpu

E = 64
K = 8
R = 16
TEMP = 0.5
EPS = 1e-6
TILE = 512
_FBASE = 0x3F800000


def _phase1_kernel(x_ref, ws_ref, h_ref, pmax_ref):
    h = jax.lax.dot_general(
        x_ref[:], ws_ref[:],
        dimension_numbers=(((1,), (1,)), ((), ())),
        preferred_element_type=jnp.float32,
    )
    h_ref[:] = h
    pmax_ref[:] = jnp.max(jnp.abs(h)).reshape(1, 1, 1)


def _phase2_kernel(x_ref, h_ref, pmax_ref, w_ref, b_ref, a_ref, bm_ref,
                   limes_ref, out_ref):
    x = x_ref[:].astype(jnp.bfloat16)
    base = jax.lax.dot_general(
        x, w_ref[:],
        dimension_numbers=(((1,), (1,)), ((), ())),
        preferred_element_type=jnp.float32,
    ) + b_ref[:]

    scale = jnp.maximum(jnp.max(pmax_ref[:]), EPS)
    inv = 1.0 / (scale * TEMP)
    logits = h_ref[:] * inv
    ii = jax.lax.broadcasted_iota(jnp.int32, logits.shape, 1)
    pbits = jax.lax.bitcast_convert_type(logits + 3.0, jnp.int32)
    keys = ((pbits - _FBASE) << 6) + (E - 1 - ii)

    kmax0 = jnp.max(keys, axis=-1, keepdims=True)
    mq = jax.lax.bitcast_convert_type(
        (kmax0 >> 6) + _FBASE, jnp.float32) - 3.0
    e = jnp.exp(logits - mq)

    wmat = jnp.zeros_like(e)
    masked = keys
    kmax = kmax0
    for k in range(K):
        if k:
            kmax = jnp.max(masked, axis=-1, keepdims=True)
        first = masked == kmax
        wmat = wmat + jnp.where(first, e, 0.0)
        masked = jnp.where(first, jnp.int32(-(2**31)), masked)

    ssum = jnp.sum(wmat, axis=-1, keepdims=True)
    w = wmat / ssum
    p_mix = jnp.dot(w, limes_ref[:], preferred_element_type=jnp.float32)

    u = jnp.dot(x, a_ref[:], preferred_element_type=jnp.float32)
    mod = (u * p_mix).astype(jnp.bfloat16)
    delta = jnp.dot(mod, bm_ref[:], preferred_element_type=jnp.float32)
    out_ref[:] = base + delta


def kernel(x, W, b, A, Bm, LiMEs):
    B, T, d_in = x.shape
    d_out = W.shape[0]
    n_tok = B * T
    nt = n_tok // TILE
    x2 = x.reshape(n_tok, d_in)
    W_bf = W.astype(jnp.bfloat16)
    A_bf = A.astype(jnp.bfloat16)
    Bm_bf = Bm.astype(jnp.bfloat16)

    h, pmax = pl.pallas_call(
        _phase1_kernel,
        grid=(nt,),
        in_specs=[
            pl.BlockSpec((TILE, d_in), lambda i: (i, 0)),
            pl.BlockSpec((E, d_in), lambda i: (0, 0)),
        ],
        out_specs=[
            pl.BlockSpec((TILE, E), lambda i: (i, 0)),
            pl.BlockSpec((1, 1, 1), lambda i: (i, 0, 0)),
        ],
        out_shape=[
            jax.ShapeDtypeStruct((n_tok, E), jnp.float32),
            jax.ShapeDtypeStruct((nt, 1, 1), jnp.float32),
        ],
        compiler_params=pltpu.CompilerParams(
            dimension_semantics=("parallel",)),
    )(x2, W)

    out = pl.pallas_call(
        _phase2_kernel,
        grid=(nt,),
        in_specs=[
            pl.BlockSpec((TILE, d_in), lambda i: (i, 0)),
            pl.BlockSpec((TILE, E), lambda i: (i, 0)),
            pl.BlockSpec((nt, 1, 1), lambda i: (0, 0, 0)),
            pl.BlockSpec((d_out, d_in), lambda i: (0, 0)),
            pl.BlockSpec((1, d_out), lambda i: (0, 0)),
            pl.BlockSpec((d_in, R), lambda i: (0, 0)),
            pl.BlockSpec((R, d_out), lambda i: (0, 0)),
            pl.BlockSpec((E, R), lambda i: (0, 0)),
        ],
        out_specs=pl.BlockSpec((TILE, d_out), lambda i: (i, 0)),
        out_shape=jax.ShapeDtypeStruct((n_tok, d_out), jnp.float32),
        compiler_params=pltpu.CompilerParams(
            dimension_semantics=("parallel",)),
    )(x2, h, pmax, W_bf, b.reshape(1, d_out), A_bf, Bm_bf, LiMEs)

    return out.reshape(B, T, d_out)


# speedup vs baseline: 1.4441x; 1.2493x over previous
import jax
import jax.numpy as jnp
from jax.experimental import pallas as pl
from jax.experimental.pallas import tpu as pltpu

E = 64
K = 8
R = 16
TEMP = 0.5
EPS = 1e-6
TILE = 1024
_FBASE = 0x3F800000


def _phase1_kernel(x_ref, ws_ref, h_ref, pmax_ref):
    h = jax.lax.dot_general(
        x_ref[:], ws_ref[:],
        dimension_numbers=(((1,), (1,)), ((), ())),
        preferred_element_type=jnp.float32,
    )
    h_ref[:] = h
    pmax_ref[:] = jnp.max(jnp.abs(h)).reshape(1, 1, 1)


def _phase2_kernel(x_ref, h_ref, pmax_ref, w_ref, b_ref, a_ref, bm_ref,
                   limes_ref, out_ref):
    x = x_ref[:].astype(jnp.bfloat16)
    base = jax.lax.dot_general(
        x, w_ref[:],
        dimension_numbers=(((1,), (1,)), ((), ())),
        preferred_element_type=jnp.float32,
    ) + b_ref[:]

    scale = jnp.maximum(jnp.max(pmax_ref[:]), EPS)
    inv = 1.0 / (scale * TEMP)
    logits = h_ref[:] * inv
    ii = jax.lax.broadcasted_iota(jnp.int32, logits.shape, 1)
    pbits = jax.lax.bitcast_convert_type(logits + 3.0, jnp.int32)
    keys = ((pbits - _FBASE) << 6) + (E - 1 - ii)

    kmax0 = jnp.max(keys, axis=-1, keepdims=True)
    mq = jax.lax.bitcast_convert_type(
        (kmax0 >> 6) + _FBASE, jnp.float32) - 3.0
    e = jnp.exp(logits - mq)

    wmat = jnp.zeros_like(e)
    masked = keys
    kmax = kmax0
    for k in range(K):
        if k:
            kmax = jnp.max(masked, axis=-1, keepdims=True)
        first = masked == kmax
        wmat = wmat + jnp.where(first, e, 0.0)
        masked = jnp.where(first, jnp.int32(-(2**31)), masked)

    ssum = jnp.sum(wmat, axis=-1, keepdims=True)
    w = wmat / ssum
    p_mix = jnp.dot(w, limes_ref[:], preferred_element_type=jnp.float32)

    u = jnp.dot(x, a_ref[:], preferred_element_type=jnp.float32)
    mod = (u * p_mix).astype(jnp.bfloat16)
    delta = jnp.dot(mod, bm_ref[:], preferred_element_type=jnp.float32)
    out_ref[:] = base + delta


def kernel(x, W, b, A, Bm, LiMEs):
    B, T, d_in = x.shape
    d_out = W.shape[0]
    n_tok = B * T
    nt = n_tok // TILE
    x2 = x.reshape(n_tok, d_in)
    W_bf = W.astype(jnp.bfloat16)
    A_bf = A.astype(jnp.bfloat16)
    Bm_bf = Bm.astype(jnp.bfloat16)

    h, pmax = pl.pallas_call(
        _phase1_kernel,
        grid=(nt,),
        in_specs=[
            pl.BlockSpec((TILE, d_in), lambda i: (i, 0)),
            pl.BlockSpec((E, d_in), lambda i: (0, 0)),
        ],
        out_specs=[
            pl.BlockSpec((TILE, E), lambda i: (i, 0)),
            pl.BlockSpec((1, 1, 1), lambda i: (i, 0, 0)),
        ],
        out_shape=[
            jax.ShapeDtypeStruct((n_tok, E), jnp.float32),
            jax.ShapeDtypeStruct((nt, 1, 1), jnp.float32),
        ],
        compiler_params=pltpu.CompilerParams(
            dimension_semantics=("parallel",)),
    )(x2, W)

    out = pl.pallas_call(
        _phase2_kernel,
        grid=(nt,),
        in_specs=[
            pl.BlockSpec((TILE, d_in), lambda i: (i, 0)),
            pl.BlockSpec((TILE, E), lambda i: (i, 0)),
            pl.BlockSpec((nt, 1, 1), lambda i: (0, 0, 0)),
            pl.BlockSpec((d_out, d_in), lambda i: (0, 0)),
            pl.BlockSpec((1, d_out), lambda i: (0, 0)),
            pl.BlockSpec((d_in, R), lambda i: (0, 0)),
            pl.BlockSpec((R, d_out), lambda i: (0, 0)),
            pl.BlockSpec((E, R), lambda i: (0, 0)),
        ],
        out_specs=pl.BlockSpec((TILE, d_out), lambda i: (i, 0)),
        out_shape=jax.ShapeDtypeStruct((n_tok, d_out), jnp.float32),
        compiler_params=pltpu.CompilerParams(
            dimension_semantics=("parallel",)),
    )(x2, h, pmax, W_bf, b.reshape(1, d_out), A_bf, Bm_bf, LiMEs)

    return out.reshape(B, T, d_out)
